# baseline (speedup 1.0000x reference)
import jax
import jax.numpy as jnp
from jax.experimental import pallas as pl

K = 12
SB = 8


def _onehot_block(x_ref, o_ref):
    x = x_ref[...]
    for k in range(K):
        o_ref[k, :, :] = (x == k).astype(jnp.float32)


def kernel(inputs):
    B, S = inputs.shape
    xt = inputs.astype(jnp.int32).T
    out_t = pl.pallas_call(
        _onehot_block,
        grid=(S // SB,),
        in_specs=[pl.BlockSpec((SB, B), lambda i: (i, 0))],
        out_specs=pl.BlockSpec((K, SB, B), lambda i: (0, i, 0)),
        out_shape=jax.ShapeDtypeStruct((K, S, B), jnp.float32),
    )(xt)
    return out_t.transpose(2, 1, 0)


# speedup vs baseline: 1.0226x; 1.0226x over previous
import jax
import jax.numpy as jnp
from jax.experimental import pallas as pl

K = 12
CI = 1024


def _onehot_block(x_ref, o_ref):
    x = x_ref[...]
    for k in range(K):
        o_ref[k, :, :] = (x == k).astype(jnp.float32)


def kernel(inputs):
    B, S = inputs.shape
    xt = inputs.astype(jnp.int32).T
    out_t = pl.pallas_call(
        _onehot_block,
        grid=(B // CI,),
        in_specs=[pl.BlockSpec((S, CI), lambda i: (0, i))],
        out_specs=pl.BlockSpec((K, S, CI), lambda i: (0, 0, i)),
        out_shape=jax.ShapeDtypeStruct((K, S, B), jnp.float32),
    )(xt)
    return out_t.transpose(2, 1, 0)
